# baseline (speedup 1.0000x reference)
import functools

import jax
import jax.numpy as jnp
from jax import lax
from jax.experimental import pallas as pl
from jax.experimental.pallas import tpu as pltpu
from jax.experimental.pallas import tpu_sc as plsc

N_SRC = 10000
N_DST = 10000
E = 320000
H = 128

NC, NS, L = 2, 16, 16
NW = NC * NS
PHASES = (83200, 83200, 76800, 76800)
GCH = 200
SCH = 200


def _sc_mesh():
    return plsc.VectorSubcoreMesh(core_axis_name="c", subcore_axis_name="s",
                                  num_cores=NC, num_subcores=NS)


def _sc_gather_add(a_tab, b_tab, src_h, dst_h, n_edges):
    ew = n_edges // NW
    nch = ew // GCH
    npair = nch // 2

    @functools.partial(
        pl.kernel, mesh=_sc_mesh(),
        out_type=jax.ShapeDtypeStruct((n_edges, H), jnp.float32),
        scratch_types=[
            pltpu.VMEM((ew,), jnp.int32),
            pltpu.VMEM((ew,), jnp.int32),
            pltpu.VMEM((GCH, H), jnp.float32),
            pltpu.VMEM((GCH, H), jnp.float32),
            pltpu.VMEM((GCH, H), jnp.float32),
            pltpu.VMEM((GCH, H), jnp.float32),
            pltpu.SemaphoreType.DMA,
            pltpu.SemaphoreType.DMA,
            pltpu.SemaphoreType.DMA,
            pltpu.SemaphoreType.DMA,
            pltpu.SemaphoreType.DMA,
            pltpu.SemaphoreType.DMA,
        ])
    def k(a_hbm, b_hbm, src_hbm, dst_hbm, out_hbm, si, di,
          ba0, bb0, ba1, bb1, sa0, sb0, sa1, sb1, so0, so1):
        wid = lax.axis_index("s") * NC + lax.axis_index("c")
        base0 = wid * ew
        pltpu.sync_copy(src_hbm.at[pl.ds(base0, ew)], si)
        pltpu.sync_copy(dst_hbm.at[pl.ds(base0, ew)], di)
        bufs = ((ba0, bb0, sa0, sb0, so0), (ba1, bb1, sa1, sb1, so1))

        def issue(ci, b):
            ba, bb, sa, sb, _ = bufs[b]
            off = ci * GCH
            pltpu.async_copy(a_hbm.at[si.at[pl.ds(off, GCH)]], ba, sa)
            pltpu.async_copy(b_hbm.at[di.at[pl.ds(off, GCH)]], bb, sb)

        def process(ci, b, last):
            ba, bb, sa, sb, so = bufs[b]
            pltpu.make_async_copy(a_hbm.at[si.at[pl.ds(0, GCH)]], ba, sa).wait()
            pltpu.make_async_copy(b_hbm.at[di.at[pl.ds(0, GCH)]], bb, sb).wait()

            def add_row(r, c2):
                for c in range(H // L):
                    sl = pl.ds(c * L, L)
                    ba[r, sl] = ba[r, sl] + bb[r, sl]
                return c2

            lax.fori_loop(0, GCH, add_row, 0)
            st = pltpu.async_copy(ba, out_hbm.at[pl.ds(base0 + ci * GCH, GCH)],
                                  so)
            if last:
                st.wait()

        issue(0, 0)
        issue(1, 1)

        def body(p, carry):
            c0 = 2 * p
            for b in range(2):
                ci = c0 + b
                process(ci, b, False)

                @pl.when(ci + 2 < nch)
                def _():
                    nxt = bufs[b]
                    pltpu.make_async_copy(
                        nxt[0], out_hbm.at[pl.ds(base0, GCH)], nxt[4]).wait()
                    issue(ci + 2, b)
            return carry

        lax.fori_loop(0, npair, body, 0)
        if nch % 2 == 1:
            process(nch - 1, 0, False)
        for b in range(2):
            pltpu.make_async_copy(
                bufs[b][0], out_hbm.at[pl.ds(base0, GCH)], bufs[b][4]).wait()

    return k(a_tab, b_tab, src_h, dst_h)


_ZSTEP = 624
_ZR = 640


def _sc_segment_sum(m, dst_h, n_edges):
    zeros = jnp.zeros((_ZR, H), jnp.float32)
    ew = n_edges // NW
    nch = ew // SCH

    @functools.partial(
        pl.kernel, mesh=_sc_mesh(),
        out_type=jax.ShapeDtypeStruct((NC, N_DST, H), jnp.float32),
        scratch_types=[
            pltpu.VMEM((SCH, H), jnp.float32),
            pltpu.VMEM((ew,), jnp.int32),
            pltpu.VMEM_SHARED((N_DST, H), jnp.float32),
            pltpu.SemaphoreType.DMA,
        ])
    def k(m_hbm, dst_hbm, z_hbm, out_hbm, buf, di, acc, sem):
        cid = lax.axis_index("c")
        sid = lax.axis_index("s")
        wid = sid * NC + cid
        base0 = wid * ew
        pltpu.sync_copy(dst_hbm.at[pl.ds(base0, ew)], di)
        pltpu.sync_copy(z_hbm, acc.at[pl.ds(sid * _ZSTEP, _ZR)])
        plsc.subcore_barrier()

        def body(ci, carry):
            off = ci * SCH
            pltpu.sync_copy(m_hbm.at[pl.ds(base0 + off, SCH)], buf)
            pltpu.sync_copy(buf, acc.at[di.at[pl.ds(off, SCH)]], add=True)
            return carry

        lax.fori_loop(0, nch, body, 0)
        plsc.subcore_barrier()
        pltpu.sync_copy(acc.at[pl.ds(sid * _ZSTEP, _ZR)],
                        out_hbm.at[cid, pl.ds(sid * _ZSTEP, _ZR)])

    return k(m, dst_h, zeros)


def _silu(x):
    return x * (1.0 / (1.0 + jnp.exp(-x)))


def _ln(x, g, b):
    m = jnp.mean(x, axis=-1, keepdims=True)
    v = jnp.mean((x - m) ** 2, axis=-1, keepdims=True)
    return (x - m) * jax.lax.rsqrt(v + 1e-5) * g + b


def _node_body(x_ref, w1_ref, b1_ref, w2_ref, b2_ref, g_ref, b_ref, wc_ref,
               h_ref, c_ref):
    x = x_ref[...]
    h = _silu(jnp.dot(x, w1_ref[...], preferred_element_type=jnp.float32)
              + b1_ref[...])
    h = jnp.dot(h, w2_ref[...], preferred_element_type=jnp.float32) + b2_ref[...]
    h = _ln(h, g_ref[...], b_ref[...])
    h_ref[...] = h
    c_ref[...] = jnp.dot(h, wc_ref[...], preferred_element_type=jnp.float32)


def _node_mlp(x, p, wc, blk=2000):
    n, d = x.shape
    grid = (n // blk,)
    full = lambda *shape: pl.BlockSpec(shape, lambda i: (0,) * len(shape))
    return pl.pallas_call(
        _node_body,
        grid=grid,
        in_specs=[
            pl.BlockSpec((blk, d), lambda i: (i, 0)),
            full(d, H), full(H), full(H, H), full(H),
            full(H), full(H), full(H, H),
        ],
        out_specs=[pl.BlockSpec((blk, H), lambda i: (i, 0)),
                   pl.BlockSpec((blk, H), lambda i: (i, 0))],
        out_shape=[jax.ShapeDtypeStruct((n, H), jnp.float32),
                   jax.ShapeDtypeStruct((n, H), jnp.float32)],
    )(x, p['W1'], p['b1'], p['W2'], p['b2'], p['g'], p['b'], wc)


def _edge_body(ea_ref, s_ref,
               we1_ref, be1_ref, we2_ref, be2_ref, ge_ref, be_ref,
               w1c_ref, b1_ref, w2_ref, b2_ref, gc_ref, bc_ref,
               m_ref):
    ea = ea_ref[...]
    t = _silu(jnp.dot(ea, we1_ref[...], preferred_element_type=jnp.float32)
              + be1_ref[...])
    e = jnp.dot(t, we2_ref[...], preferred_element_type=jnp.float32) + be2_ref[...]
    e = _ln(e, ge_ref[...], be_ref[...])
    u = s_ref[...] + jnp.dot(e, w1c_ref[...], preferred_element_type=jnp.float32) \
        + b1_ref[...]
    u = _silu(u)
    m = jnp.dot(u, w2_ref[...], preferred_element_type=jnp.float32) + b2_ref[...]
    m_ref[...] = _ln(m, gc_ref[...], bc_ref[...])


def _edge_mlp(edge_attr, s_h, e_off, pe, w1c, b1, w2, b2, gc, bc, blk=6400):
    e_dim = edge_attr.shape[1]
    n_edges = s_h.shape[0]
    grid = (n_edges // blk,)
    off = e_off // blk
    full = lambda *shape: pl.BlockSpec(shape, lambda i: (0,) * len(shape))
    return pl.pallas_call(
        _edge_body,
        grid=grid,
        in_specs=[
            pl.BlockSpec((blk, e_dim), lambda i: (i + off, 0)),
            pl.BlockSpec((blk, H), lambda i: (i, 0)),
            full(e_dim, H), full(H), full(H, H), full(H), full(H), full(H),
            full(H, H), full(H), full(H, H), full(H), full(H), full(H),
        ],
        out_specs=pl.BlockSpec((blk, H), lambda i: (i, 0)),
        out_shape=jax.ShapeDtypeStruct((n_edges, H), jnp.float32),
    )(edge_attr, s_h, pe['W1'], pe['b1'], pe['W2'], pe['b2'], pe['g'], pe['b'],
      w1c, b1, w2, b2, gc, bc)


def _post_body(hd_ref, *refs):
    agg_refs = refs[:-5]
    w1_ref, b1_ref, w2_ref, b2_ref, o_ref = refs[-5:]
    x = hd_ref[...]
    for a in agg_refs:
        x = x + a[0] + a[1]
    h = _silu(jnp.dot(x, w1_ref[...], preferred_element_type=jnp.float32)
              + b1_ref[...])
    o_ref[...] = jnp.dot(h, w2_ref[...], preferred_element_type=jnp.float32) \
        + b2_ref[...]


def _post_mlp(hd, aggs, p, blk=2000):
    n = hd.shape[0]
    out = p['W2'].shape[1]
    grid = (n // blk,)
    full = lambda *shape: pl.BlockSpec(shape, lambda i: (0,) * len(shape))
    return pl.pallas_call(
        _post_body,
        grid=grid,
        in_specs=[
            pl.BlockSpec((blk, H), lambda i: (i, 0)),
            *[pl.BlockSpec((NC, blk, H), lambda i: (0, i, 0)) for _ in aggs],
            full(H, H), full(H), full(H, out), full(out),
        ],
        out_specs=pl.BlockSpec((blk, out), lambda i: (i, 0)),
        out_shape=jax.ShapeDtypeStruct((n, out), jnp.float32),
    )(hd, *aggs, p['W1'], p['b1'], p['W2'], p['b2'])


def kernel(x_src, x_dst, batch_size, edge_attr, edge_index, params):
    pc = params['conv']
    w1a = pc['W1'][:H]
    w1b = pc['W1'][H:2 * H]
    w1c = pc['W1'][2 * H:]

    hs, a_tab = _node_mlp(x_src, params['emb_src'], w1a)
    hd, b_tab = _node_mlp(x_dst, params['emb_dst'], w1b)

    src, dst = edge_index[0], edge_index[1]

    aggs = []
    e_off = 0
    for eph in PHASES:
        src_h = lax.slice_in_dim(src, e_off, e_off + eph)
        dst_h = lax.slice_in_dim(dst, e_off, e_off + eph)
        s_h = _sc_gather_add(a_tab, b_tab, src_h, dst_h, eph)
        m_h = _edge_mlp(edge_attr, s_h, e_off, params['emb_edges'],
                        w1c, pc['b1'], pc['W2'], pc['b2'], pc['g'], pc['b'])
        aggs.append(_sc_segment_sum(m_h, dst_h, eph))
        e_off += eph

    out_dst = _post_mlp(hd, aggs, params['post'])
    return hs, out_dst


# speedup vs baseline: 1.0071x; 1.0071x over previous
import functools

import jax
import jax.numpy as jnp
from jax import lax
from jax.experimental import pallas as pl
from jax.experimental.pallas import tpu as pltpu
from jax.experimental.pallas import tpu_sc as plsc

N_SRC = 10000
N_DST = 10000
E = 320000
H = 128

NC, NS, L = 2, 16, 16
NW = NC * NS
PHASES = (83200, 83200, 76800, 76800)
GCH = 200
SCH = 200


def _sc_mesh():
    return plsc.VectorSubcoreMesh(core_axis_name="c", subcore_axis_name="s",
                                  num_cores=NC, num_subcores=NS)


def _sc_gather_add(a_tab, b_tab, src_h, dst_h, n_edges):
    ew = n_edges // NW
    nch = ew // GCH
    npair = nch // 2

    @functools.partial(
        pl.kernel, mesh=_sc_mesh(),
        out_type=jax.ShapeDtypeStruct((n_edges, H), jnp.float32),
        scratch_types=[
            pltpu.VMEM((ew,), jnp.int32),
            pltpu.VMEM((ew,), jnp.int32),
            pltpu.VMEM((GCH, H), jnp.float32),
            pltpu.VMEM((GCH, H), jnp.float32),
            pltpu.VMEM((GCH, H), jnp.float32),
            pltpu.VMEM((GCH, H), jnp.float32),
            pltpu.SemaphoreType.DMA,
            pltpu.SemaphoreType.DMA,
            pltpu.SemaphoreType.DMA,
            pltpu.SemaphoreType.DMA,
            pltpu.SemaphoreType.DMA,
            pltpu.SemaphoreType.DMA,
        ])
    def k(a_hbm, b_hbm, src_hbm, dst_hbm, out_hbm, si, di,
          ba0, bb0, ba1, bb1, sa0, sb0, sa1, sb1, so0, so1):
        wid = lax.axis_index("s") * NC + lax.axis_index("c")
        base0 = wid * ew
        pltpu.sync_copy(src_hbm.at[pl.ds(base0, ew)], si)
        pltpu.sync_copy(dst_hbm.at[pl.ds(base0, ew)], di)
        bufs = ((ba0, bb0, sa0, sb0, so0), (ba1, bb1, sa1, sb1, so1))

        def issue(ci, b):
            ba, bb, sa, sb, _ = bufs[b]
            off = ci * GCH
            pltpu.async_copy(a_hbm.at[si.at[pl.ds(off, GCH)]], ba, sa)
            pltpu.async_copy(b_hbm.at[di.at[pl.ds(off, GCH)]], bb, sb)

        def process(ci, b, last):
            ba, bb, sa, sb, so = bufs[b]
            pltpu.make_async_copy(a_hbm.at[si.at[pl.ds(0, GCH)]], ba, sa).wait()
            pltpu.make_async_copy(b_hbm.at[di.at[pl.ds(0, GCH)]], bb, sb).wait()

            def add_row(r, c2):
                for c in range(H // L):
                    sl = pl.ds(c * L, L)
                    ba[r, sl] = ba[r, sl] + bb[r, sl]
                return c2

            lax.fori_loop(0, GCH, add_row, 0)
            st = pltpu.async_copy(ba, out_hbm.at[pl.ds(base0 + ci * GCH, GCH)],
                                  so)
            if last:
                st.wait()

        issue(0, 0)
        issue(1, 1)

        def body(p, carry):
            c0 = 2 * p
            for b in range(2):
                ci = c0 + b
                process(ci, b, False)

                @pl.when(ci + 2 < nch)
                def _():
                    nxt = bufs[b]
                    pltpu.make_async_copy(
                        nxt[0], out_hbm.at[pl.ds(base0, GCH)], nxt[4]).wait()
                    issue(ci + 2, b)
            return carry

        lax.fori_loop(0, npair, body, 0)
        if nch % 2 == 1:
            process(nch - 1, 0, False)
        for b in range(2):
            pltpu.make_async_copy(
                bufs[b][0], out_hbm.at[pl.ds(base0, GCH)], bufs[b][4]).wait()

    return k(a_tab, b_tab, src_h, dst_h)


_ZSTEP = 624
_ZR = 640


def _sc_segment_sum(m_list, dst_list):
    zeros = jnp.zeros((_ZR, H), jnp.float32)
    npart = len(m_list)
    ew = m_list[0].shape[0] // NW
    nch = ew // SCH

    @functools.partial(
        pl.kernel, mesh=_sc_mesh(),
        out_type=jax.ShapeDtypeStruct((NC, N_DST, H), jnp.float32),
        scratch_types=[
            pltpu.VMEM((SCH, H), jnp.float32),
            pltpu.VMEM((npart * ew,), jnp.int32),
            pltpu.VMEM_SHARED((N_DST, H), jnp.float32),
            pltpu.SemaphoreType.DMA,
        ])
    def k(*refs):
        m_hbms = refs[:npart]
        dst_hbms = refs[npart:2 * npart]
        z_hbm = refs[2 * npart]
        out_hbm = refs[2 * npart + 1]
        buf, di, acc, sem = refs[2 * npart + 2:]
        cid = lax.axis_index("c")
        sid = lax.axis_index("s")
        wid = sid * NC + cid
        base0 = wid * ew
        for j, dst_hbm in enumerate(dst_hbms):
            pltpu.sync_copy(dst_hbm.at[pl.ds(base0, ew)],
                            di.at[pl.ds(j * ew, ew)])
        pltpu.sync_copy(z_hbm, acc.at[pl.ds(sid * _ZSTEP, _ZR)])
        plsc.subcore_barrier()

        for j, m_hbm in enumerate(m_hbms):
            def body(ci, carry, m_hbm=m_hbm, j=j):
                off = ci * SCH
                pltpu.sync_copy(m_hbm.at[pl.ds(base0 + off, SCH)], buf)
                pltpu.sync_copy(
                    buf, acc.at[di.at[pl.ds(j * ew + off, SCH)]], add=True)
                return carry

            lax.fori_loop(0, nch, body, 0)
        plsc.subcore_barrier()
        pltpu.sync_copy(acc.at[pl.ds(sid * _ZSTEP, _ZR)],
                        out_hbm.at[cid, pl.ds(sid * _ZSTEP, _ZR)])

    return k(*m_list, *dst_list, zeros)


def _silu(x):
    return x * (1.0 / (1.0 + jnp.exp(-x)))


def _ln(x, g, b):
    m = jnp.mean(x, axis=-1, keepdims=True)
    v = jnp.mean((x - m) ** 2, axis=-1, keepdims=True)
    return (x - m) * jax.lax.rsqrt(v + 1e-5) * g + b


def _node_body(x_ref, w1_ref, b1_ref, w2_ref, b2_ref, g_ref, b_ref, wc_ref,
               h_ref, c_ref):
    x = x_ref[...]
    h = _silu(jnp.dot(x, w1_ref[...], preferred_element_type=jnp.float32)
              + b1_ref[...])
    h = jnp.dot(h, w2_ref[...], preferred_element_type=jnp.float32) + b2_ref[...]
    h = _ln(h, g_ref[...], b_ref[...])
    h_ref[...] = h
    c_ref[...] = jnp.dot(h, wc_ref[...], preferred_element_type=jnp.float32)


def _node_mlp(x, p, wc, blk=2000):
    n, d = x.shape
    grid = (n // blk,)
    full = lambda *shape: pl.BlockSpec(shape, lambda i: (0,) * len(shape))
    return pl.pallas_call(
        _node_body,
        grid=grid,
        in_specs=[
            pl.BlockSpec((blk, d), lambda i: (i, 0)),
            full(d, H), full(H), full(H, H), full(H),
            full(H), full(H), full(H, H),
        ],
        out_specs=[pl.BlockSpec((blk, H), lambda i: (i, 0)),
                   pl.BlockSpec((blk, H), lambda i: (i, 0))],
        out_shape=[jax.ShapeDtypeStruct((n, H), jnp.float32),
                   jax.ShapeDtypeStruct((n, H), jnp.float32)],
    )(x, p['W1'], p['b1'], p['W2'], p['b2'], p['g'], p['b'], wc)


def _edge_body(ea_ref, s_ref,
               we1_ref, be1_ref, we2_ref, be2_ref, ge_ref, be_ref,
               w1c_ref, b1_ref, w2_ref, b2_ref, gc_ref, bc_ref,
               m_ref):
    ea = ea_ref[...]
    t = _silu(jnp.dot(ea, we1_ref[...], preferred_element_type=jnp.float32)
              + be1_ref[...])
    e = jnp.dot(t, we2_ref[...], preferred_element_type=jnp.float32) + be2_ref[...]
    e = _ln(e, ge_ref[...], be_ref[...])
    u = s_ref[...] + jnp.dot(e, w1c_ref[...], preferred_element_type=jnp.float32) \
        + b1_ref[...]
    u = _silu(u)
    m = jnp.dot(u, w2_ref[...], preferred_element_type=jnp.float32) + b2_ref[...]
    m_ref[...] = _ln(m, gc_ref[...], bc_ref[...])


def _edge_mlp(edge_attr, s_h, e_off, pe, w1c, b1, w2, b2, gc, bc, blk=6400):
    e_dim = edge_attr.shape[1]
    n_edges = s_h.shape[0]
    grid = (n_edges // blk,)
    off = e_off // blk
    full = lambda *shape: pl.BlockSpec(shape, lambda i: (0,) * len(shape))
    return pl.pallas_call(
        _edge_body,
        grid=grid,
        in_specs=[
            pl.BlockSpec((blk, e_dim), lambda i: (i + off, 0)),
            pl.BlockSpec((blk, H), lambda i: (i, 0)),
            full(e_dim, H), full(H), full(H, H), full(H), full(H), full(H),
            full(H, H), full(H), full(H, H), full(H), full(H), full(H),
        ],
        out_specs=pl.BlockSpec((blk, H), lambda i: (i, 0)),
        out_shape=jax.ShapeDtypeStruct((n_edges, H), jnp.float32),
    )(edge_attr, s_h, pe['W1'], pe['b1'], pe['W2'], pe['b2'], pe['g'], pe['b'],
      w1c, b1, w2, b2, gc, bc)


def _post_body(hd_ref, *refs):
    agg_refs = refs[:-5]
    w1_ref, b1_ref, w2_ref, b2_ref, o_ref = refs[-5:]
    x = hd_ref[...]
    for a in agg_refs:
        x = x + a[0] + a[1]
    h = _silu(jnp.dot(x, w1_ref[...], preferred_element_type=jnp.float32)
              + b1_ref[...])
    o_ref[...] = jnp.dot(h, w2_ref[...], preferred_element_type=jnp.float32) \
        + b2_ref[...]


def _post_mlp(hd, aggs, p, blk=2000):
    n = hd.shape[0]
    out = p['W2'].shape[1]
    grid = (n // blk,)
    full = lambda *shape: pl.BlockSpec(shape, lambda i: (0,) * len(shape))
    return pl.pallas_call(
        _post_body,
        grid=grid,
        in_specs=[
            pl.BlockSpec((blk, H), lambda i: (i, 0)),
            *[pl.BlockSpec((NC, blk, H), lambda i: (0, i, 0)) for _ in aggs],
            full(H, H), full(H), full(H, out), full(out),
        ],
        out_specs=pl.BlockSpec((blk, out), lambda i: (i, 0)),
        out_shape=jax.ShapeDtypeStruct((n, out), jnp.float32),
    )(hd, *aggs, p['W1'], p['b1'], p['W2'], p['b2'])


def kernel(x_src, x_dst, batch_size, edge_attr, edge_index, params):
    pc = params['conv']
    w1a = pc['W1'][:H]
    w1b = pc['W1'][H:2 * H]
    w1c = pc['W1'][2 * H:]

    hs, a_tab = _node_mlp(x_src, params['emb_src'], w1a)
    hd, b_tab = _node_mlp(x_dst, params['emb_dst'], w1b)

    src, dst = edge_index[0], edge_index[1]

    ms, dsts = [], []
    e_off = 0
    for eph in PHASES:
        src_h = lax.slice_in_dim(src, e_off, e_off + eph)
        dst_h = lax.slice_in_dim(dst, e_off, e_off + eph)
        s_h = _sc_gather_add(a_tab, b_tab, src_h, dst_h, eph)
        ms.append(_edge_mlp(edge_attr, s_h, e_off, params['emb_edges'],
                            w1c, pc['b1'], pc['W2'], pc['b2'], pc['g'], pc['b']))
        dsts.append(dst_h)
        e_off += eph

    aggs = [_sc_segment_sum(ms[0:2], dsts[0:2]),
            _sc_segment_sum(ms[2:4], dsts[2:4])]

    out_dst = _post_mlp(hd, aggs, params['post'])
    return hs, out_dst
